# Initial kernel scaffold; baseline (speedup 1.0000x reference)
import jax
import jax.numpy as jnp
from jax.experimental import pallas as pl


def kernel(x, W1, b1, W2, b2, W3, b3, W4, b4, emb):
    raise NotImplementedError("write your pallas kernel here")


# speedup vs baseline: 4.1861x; 4.1861x over previous
import numpy as np

import jax
import jax.numpy as jnp
from jax.experimental import pallas as pl
from jax.experimental.pallas import tpu as pltpu

TOKENS = 2048
DIM = 1024
HIDDEN = 4096
NUM_EMB = 4096

PREC = jax.lax.Precision.DEFAULT

_M = np.uint32(73244475)
_ML = np.uint32(73244475 & 0xFFFF)
_MH = np.uint32(73244475 >> 16)
_U16 = np.uint32(16)


def _dot(a, b, prec=None):
    return jax.lax.dot_general(
        a, b, (((1,), (0,)), ((), ())),
        precision=(prec or PREC), preferred_element_type=jnp.float32)


def _dot_t(a, b, prec=None):
    return jax.lax.dot_general(
        a, b, (((1,), (1,)), ((), ())),
        precision=(prec or PREC), preferred_element_type=jnp.float32)


def _fourier(s):
    lane = jax.lax.broadcasted_iota(jnp.int32, s.shape, len(s.shape) - 1)
    return jnp.where((lane & 1) == 0, jnp.sin(s), jnp.cos(s))


def _hash_mod(v_u32):
    p = v_u32 * _ML
    q = v_u32 * _MH
    lo = p + (q << _U16)
    carry = (lo < p).astype(jnp.uint32)
    hi = (q >> _U16) + carry
    t = (lo >> _U16) | ((hi & np.uint32(0xFFFF)) << _U16)
    y1 = lo ^ t
    x2 = y1 * _M
    return ((x2 ^ (x2 >> _U16)) & np.uint32(0xFFF)).astype(jnp.int32)


def _k1_body(x_ref, w1_ref, b1_ref, p_ref, f_ref):
    s = _dot(x_ref[...], w1_ref[...]) + b1_ref[...]
    p_ref[...] = s
    f_ref[...] = _fourier(s)


def _k2_body(p_ref, f_ref, w2_ref, b2_ref, o_ref, acc):
    k = pl.program_id(2)
    nk = pl.num_programs(2)
    half = nk // 2

    @pl.when(k == 0)
    def _():
        acc[...] = jnp.zeros_like(acc)

    @pl.when(k < half)
    def _():
        acc[...] += _dot(p_ref[...], w2_ref[...])

    @pl.when(k >= half)
    def _():
        acc[...] += _dot(f_ref[...], w2_ref[...])

    @pl.when(k == nk - 1)
    def _():
        o_ref[...] = _fourier(acc[...] + b2_ref[...])


def _k3_body(h_ref, w3_ref, b3_ref, o_ref):
    o_ref[...] = _fourier(_dot(h_ref[...], w3_ref[...]) + b3_ref[...])


def _k4_body(h_ref, w4_ref, b4_ref, o_ref):
    o_ref[...] = _dot(h_ref[...], w4_ref[...]) + b4_ref[...]


def _e2_body(emb_ref, o_ref):
    e = emb_ref[...]
    o_ref[...] = _dot_t(jnp.ones((8, DIM), jnp.float32), e * e,
                        prec=jax.lax.Precision.HIGHEST)


def _k5_body(lat_ref, emb_ref, e2_ref, idx_ref):
    lat = lat_ref[...]
    x2 = jnp.sum(lat * lat, axis=-1, keepdims=True)
    xe = _dot_t(lat, emb_ref[...])
    d2 = x2 + e2_ref[...] - 2.0 * xe
    dist = jnp.sqrt(jnp.maximum(d2, 0.0))
    m = jnp.min(dist, axis=-1, keepdims=True)
    lane = jax.lax.broadcasted_iota(jnp.int32, dist.shape, 1)
    idx = jnp.min(jnp.where(dist == m, lane, NUM_EMB), axis=-1, keepdims=True)
    idx_ref[...] = idx


def _k6_body(idx_ref, emb_ref, o_ref):
    idx = idx_ref[...].astype(jnp.uint32)
    lane = jax.lax.broadcasted_iota(jnp.int32, (idx.shape[0], NUM_EMB), 1)
    oh = jnp.zeros((idx.shape[0], NUM_EMB), jnp.float32)
    for i in range(1, 4):
        sel = _hash_mod(idx + np.uint32(i * NUM_EMB))
        oh = oh + (lane == sel).astype(jnp.float32)
    o_ref[...] = _dot(oh, emb_ref[...], prec=jax.lax.Precision.HIGHEST) / 3.0


@jax.jit
def kernel(x, W1, b1, W2, b2, W3, b3, W4, b4, emb):
    with jax.enable_x64(False):
        return _impl(x, W1, b1, W2, b2, W3, b3, W4, b4, emb)


def _impl(x, W1, b1, W2, b2, W3, b3, W4, b4, emb):
    f32 = jnp.float32
    bm = 512
    bn = 512
    nm = TOKENS // bm

    b1r = b1.reshape(1, HIDDEN)
    b2r = b2.reshape(1, HIDDEN)
    b3r = b3.reshape(1, HIDDEN)
    b4r = b4.reshape(1, DIM)

    p1, f1 = pl.pallas_call(
        _k1_body,
        grid=(nm, HIDDEN // bn),
        in_specs=[
            pl.BlockSpec((bm, DIM), lambda m, n: (m, 0)),
            pl.BlockSpec((DIM, bn), lambda m, n: (0, n)),
            pl.BlockSpec((1, bn), lambda m, n: (0, n)),
        ],
        out_specs=[
            pl.BlockSpec((bm, bn), lambda m, n: (m, n)),
            pl.BlockSpec((bm, bn), lambda m, n: (m, n)),
        ],
        out_shape=[
            jax.ShapeDtypeStruct((TOKENS, HIDDEN), f32),
            jax.ShapeDtypeStruct((TOKENS, HIDDEN), f32),
        ],
        compiler_params=pltpu.CompilerParams(
            dimension_semantics=("parallel", "parallel")),
    )(x, W1, b1r)

    bk = 1024
    nkh = HIDDEN // bk
    h2 = pl.pallas_call(
        _k2_body,
        grid=(nm, HIDDEN // bn, 2 * nkh),
        in_specs=[
            pl.BlockSpec((bm, bk), lambda m, n, k: (m, jnp.minimum(k, nkh - 1))),
            pl.BlockSpec((bm, bk), lambda m, n, k: (m, jnp.maximum(k - nkh, 0))),
            pl.BlockSpec((bk, bn), lambda m, n, k: (k, n)),
            pl.BlockSpec((1, bn), lambda m, n, k: (0, n)),
        ],
        out_specs=pl.BlockSpec((bm, bn), lambda m, n, k: (m, n)),
        out_shape=jax.ShapeDtypeStruct((TOKENS, HIDDEN), f32),
        scratch_shapes=[pltpu.VMEM((bm, bn), f32)],
        compiler_params=pltpu.CompilerParams(
            dimension_semantics=("parallel", "parallel", "arbitrary")),
    )(p1, f1, W2, b2r)

    h3 = pl.pallas_call(
        _k3_body,
        grid=(nm, HIDDEN // bn),
        in_specs=[
            pl.BlockSpec((bm, HIDDEN), lambda m, n: (m, 0)),
            pl.BlockSpec((HIDDEN, bn), lambda m, n: (0, n)),
            pl.BlockSpec((1, bn), lambda m, n: (0, n)),
        ],
        out_specs=pl.BlockSpec((bm, bn), lambda m, n: (m, n)),
        out_shape=jax.ShapeDtypeStruct((TOKENS, HIDDEN), f32),
        compiler_params=pltpu.CompilerParams(
            dimension_semantics=("parallel", "parallel")),
    )(h2, W3, b3r)

    lat = pl.pallas_call(
        _k4_body,
        grid=(nm, DIM // bn),
        in_specs=[
            pl.BlockSpec((bm, HIDDEN), lambda m, n: (m, 0)),
            pl.BlockSpec((HIDDEN, bn), lambda m, n: (0, n)),
            pl.BlockSpec((1, bn), lambda m, n: (0, n)),
        ],
        out_specs=pl.BlockSpec((bm, bn), lambda m, n: (m, n)),
        out_shape=jax.ShapeDtypeStruct((TOKENS, DIM), f32),
        compiler_params=pltpu.CompilerParams(
            dimension_semantics=("parallel", "parallel")),
    )(h3, W4, b4r)

    e2 = jnp.sum(emb * emb, axis=-1)[None, :]

    idx = pl.pallas_call(
        _k5_body,
        grid=(nm,),
        in_specs=[
            pl.BlockSpec((bm, DIM), lambda m: (m, 0)),
            pl.BlockSpec((NUM_EMB, DIM), lambda m: (0, 0)),
            pl.BlockSpec((1, NUM_EMB), lambda m: (0, 0)),
        ],
        out_specs=pl.BlockSpec((bm, 1), lambda m: (m, 0)),
        out_shape=jax.ShapeDtypeStruct((TOKENS, 1), jnp.int32),
        compiler_params=pltpu.CompilerParams(
            dimension_semantics=("parallel",)),
    )(lat, emb, e2)

    out = pl.pallas_call(
        _k6_body,
        grid=(nm,),
        in_specs=[
            pl.BlockSpec((bm, 1), lambda m: (m, 0)),
            pl.BlockSpec((NUM_EMB, DIM), lambda m: (0, 0)),
        ],
        out_specs=pl.BlockSpec((bm, DIM), lambda m: (m, 0)),
        out_shape=jax.ShapeDtypeStruct((TOKENS, DIM), f32),
        compiler_params=pltpu.CompilerParams(
            dimension_semantics=("parallel",)),
    )(idx, emb)

    return out
